# baseline (speedup 1.0000x reference)
import functools

import jax
import jax.numpy as jnp
from jax import lax
from jax.experimental import pallas as pl
from jax.experimental.pallas import tpu as pltpu

_NBUF = 4
_CH = 512


def _body(x_hbm, pk_ref, idx_ref, sim_ref, bkn_ref, rs_ref,
          buf, xmax_ref, sem, *, batch, seq, pool, topk, d_model):
    cpb = seq // _CH
    nchunks = batch * cpb

    def _copy(c, ib):
        b = c // cpb
        j = lax.rem(c, cpb)
        return pltpu.make_async_copy(
            x_hbm.at[b, pl.ds(j * _CH, _CH), :],
            buf.at[ib], sem.at[ib])

    for i in range(_NBUF - 1):
        _copy(i, i).start()

    def step(c, _):
        ib = lax.rem(c, _NBUF)
        b = c // cpb
        j = lax.rem(c, cpb)
        _copy(c, ib).wait()

        nxt = c + _NBUF - 1
        @pl.when(nxt < nchunks)
        def _refill():
            _copy(nxt, lax.rem(nxt, _NBUF)).start()

        xm = jnp.max(buf[ib], axis=0)[None, :]

        @pl.when(j == 0)
        def _init():
            xmax_ref[pl.ds(b, 1), :] = xm

        @pl.when(j != 0)
        def _acc():
            xmax_ref[pl.ds(b, 1), :] = jnp.maximum(xmax_ref[pl.ds(b, 1), :],
                                                   xm)

        return 0

    lax.fori_loop(0, nchunks, step, 0)

    xmax = xmax_ref[0:batch, :]
    pk = pk_ref[...]
    pn = pk * jax.lax.rsqrt(
        jnp.maximum(jnp.sum(pk * pk, axis=1, keepdims=True), 1e-12))
    xn = xmax * jax.lax.rsqrt(
        jnp.maximum(jnp.sum(xmax * xmax, axis=1, keepdims=True), 1e-12))
    sim = jax.lax.dot_general(
        xn, pn, (((1,), (1,)), ((), ())),
        preferred_element_type=jnp.float32)
    iota = jax.lax.broadcasted_iota(jnp.int32, (batch, pool), 1)
    big = jnp.int32(pool)
    neg = jnp.float32(-jnp.inf)
    v1 = jnp.max(sim, axis=1, keepdims=True)
    i1 = jnp.min(jnp.where(sim == v1, iota, big), axis=1, keepdims=True)
    sim2 = jnp.where(iota == i1, neg, sim)
    v2 = jnp.max(sim2, axis=1, keepdims=True)
    i2 = jnp.min(jnp.where(sim2 == v2, iota, big), axis=1, keepdims=True)
    idx_ref[...] = jnp.concatenate([i1, i2], axis=1)
    oh1 = (iota == i1).astype(jnp.float32)
    oh2 = (iota == i2).astype(jnp.float32)
    bkn1 = jax.lax.dot_general(
        oh1, pn, (((1,), (0,)), ((), ())),
        preferred_element_type=jnp.float32)
    bkn2 = jax.lax.dot_general(
        oh2, pn, (((1,), (0,)), ((), ())),
        preferred_element_type=jnp.float32)
    bkn_ref[...] = jnp.concatenate(
        [bkn1[:, None, :], bkn2[:, None, :]], axis=1)
    sim_ref[...] = sim
    rs_ref[...] = ((jnp.sum(v1) + jnp.sum(v2)) / batch)[None, None]


def kernel(x_embed, prompt_key):
    batch, seq, d_model = x_embed.shape
    pool = prompt_key.shape[0]
    topk = 2

    out = pl.pallas_call(
        functools.partial(_body, batch=batch, seq=seq, pool=pool, topk=topk,
                          d_model=d_model),
        in_specs=[
            pl.BlockSpec(memory_space=pl.ANY),
            pl.BlockSpec((pool, d_model), lambda: (0, 0)),
        ],
        out_specs=[
            pl.BlockSpec((batch, topk), lambda: (0, 0)),
            pl.BlockSpec((batch, pool), lambda: (0, 0)),
            pl.BlockSpec((batch, topk, d_model), lambda: (0, 0, 0)),
            pl.BlockSpec((1, 1), lambda: (0, 0)),
        ],
        out_shape=[
            jax.ShapeDtypeStruct((batch, topk), jnp.int32),
            jax.ShapeDtypeStruct((batch, pool), jnp.float32),
            jax.ShapeDtypeStruct((batch, topk, d_model), jnp.float32),
            jax.ShapeDtypeStruct((1, 1), jnp.float32),
        ],
        scratch_shapes=[
            pltpu.VMEM((_NBUF, _CH, d_model), jnp.float32),
            pltpu.VMEM((max(batch, 8), d_model), jnp.float32),
            pltpu.SemaphoreType.DMA((_NBUF,)),
        ],
    )(x_embed, prompt_key)

    idx, sim, bkn, rs = out
    return (idx, sim, bkn, rs.reshape(()))


# speedup vs baseline: 1.1180x; 1.1180x over previous
import functools

import jax
import jax.numpy as jnp
from jax.experimental import pallas as pl
from jax.experimental.pallas import tpu as pltpu


def _body(x1_ref, x2_ref, pk_ref, idx_ref, sim_ref, bkn_ref, rs_ref,
          xmax_ref, *, batch, pool, topk, d_model, nstep):
    s = pl.program_id(0)
    bp = batch // nstep
    xm = jnp.maximum(jnp.max(x1_ref[...], axis=1),
                     jnp.max(x2_ref[...], axis=1))
    xmax_ref[pl.ds(pl.multiple_of(s * 8, 8), bp), :] = xm

    @pl.when(s == nstep - 1)
    def _tail():
        xmax = jnp.concatenate(
            [xmax_ref[i * 8:i * 8 + bp, :] for i in range(nstep)],
            axis=0)
        pk = pk_ref[...]
        pn = pk * jax.lax.rsqrt(
            jnp.maximum(jnp.sum(pk * pk, axis=1, keepdims=True), 1e-12))
        xn = xmax * jax.lax.rsqrt(
            jnp.maximum(jnp.sum(xmax * xmax, axis=1, keepdims=True), 1e-12))
        sim = jax.lax.dot_general(
            xn, pn, (((1,), (1,)), ((), ())),
            preferred_element_type=jnp.float32)
        iota = jax.lax.broadcasted_iota(jnp.int32, (batch, pool), 1)
        big = jnp.int32(pool)
        neg = jnp.float32(-jnp.inf)
        v1 = jnp.max(sim, axis=1, keepdims=True)
        i1 = jnp.min(jnp.where(sim == v1, iota, big), axis=1, keepdims=True)
        sim2 = jnp.where(iota == i1, neg, sim)
        v2 = jnp.max(sim2, axis=1, keepdims=True)
        i2 = jnp.min(jnp.where(sim2 == v2, iota, big), axis=1, keepdims=True)
        idx = jnp.concatenate([i1, i2], axis=1)
        oh1 = (iota == i1).astype(jnp.float32)
        oh2 = (iota == i2).astype(jnp.float32)
        bkn1 = jax.lax.dot_general(
            oh1, pn, (((1,), (0,)), ((), ())),
            preferred_element_type=jnp.float32)
        bkn2 = jax.lax.dot_general(
            oh2, pn, (((1,), (0,)), ((), ())),
            preferred_element_type=jnp.float32)
        idx_ref[...] = idx
        sim_ref[...] = sim
        bkn_ref[...] = jnp.concatenate(
            [bkn1[:, None, :], bkn2[:, None, :]], axis=1)
        rs_ref[...] = ((jnp.sum(v1) + jnp.sum(v2)) / batch)[None, None]


def kernel(x_embed, prompt_key):
    batch, seq, d_model = x_embed.shape
    pool = prompt_key.shape[0]
    topk = 2
    nstep = 2
    bp = batch // nstep
    hs = seq // 2

    out = pl.pallas_call(
        functools.partial(_body, batch=batch, pool=pool, topk=topk,
                          d_model=d_model, nstep=nstep),
        grid=(nstep,),
        in_specs=[
            pl.BlockSpec((bp, hs, d_model), lambda s: (s, 0, 0)),
            pl.BlockSpec((bp, hs, d_model), lambda s: (s, 1, 0)),
            pl.BlockSpec((pool, d_model), lambda s: (0, 0)),
        ],
        out_specs=[
            pl.BlockSpec((batch, topk), lambda s: (0, 0)),
            pl.BlockSpec((batch, pool), lambda s: (0, 0)),
            pl.BlockSpec((batch, topk, d_model), lambda s: (0, 0, 0)),
            pl.BlockSpec((1, 1), lambda s: (0, 0)),
        ],
        out_shape=[
            jax.ShapeDtypeStruct((batch, topk), jnp.int32),
            jax.ShapeDtypeStruct((batch, pool), jnp.float32),
            jax.ShapeDtypeStruct((batch, topk, d_model), jnp.float32),
            jax.ShapeDtypeStruct((1, 1), jnp.float32),
        ],
        scratch_shapes=[pltpu.VMEM((nstep * 8, d_model), jnp.float32)],
        compiler_params=pltpu.CompilerParams(
            vmem_limit_bytes=100 * 1024 * 1024),
    )(x_embed, x_embed, prompt_key)

    idx, sim, bkn, rs = out
    return (idx, sim, bkn, rs.reshape(()))


# speedup vs baseline: 1.1449x; 1.0241x over previous
import functools

import jax
import jax.numpy as jnp
from jax.experimental import pallas as pl
from jax.experimental.pallas import tpu as pltpu


def _body(x1_ref, x2_ref, pk_ref, idx_ref, sim_ref, bkn_ref, rs_ref,
          xmax_ref, *, batch, pool, topk, d_model):
    b = pl.program_id(0)
    xm = jnp.maximum(jnp.max(x1_ref[0], axis=0), jnp.max(x2_ref[0], axis=0))
    xmax_ref[pl.ds(b, 1), :] = xm[None, :]

    @pl.when(b == batch - 1)
    def _tail():
        xmax = xmax_ref[0:batch, :]
        pk = pk_ref[...]
        pn = pk * jax.lax.rsqrt(
            jnp.maximum(jnp.sum(pk * pk, axis=1, keepdims=True), 1e-12))
        xn = xmax * jax.lax.rsqrt(
            jnp.maximum(jnp.sum(xmax * xmax, axis=1, keepdims=True), 1e-12))
        sim = jax.lax.dot_general(
            xn, pn, (((1,), (1,)), ((), ())),
            preferred_element_type=jnp.float32)
        iota = jax.lax.broadcasted_iota(jnp.int32, (batch, pool), 1)
        big = jnp.int32(pool)
        neg = jnp.float32(-jnp.inf)
        v1 = jnp.max(sim, axis=1, keepdims=True)
        i1 = jnp.min(jnp.where(sim == v1, iota, big), axis=1, keepdims=True)
        sim2 = jnp.where(iota == i1, neg, sim)
        v2 = jnp.max(sim2, axis=1, keepdims=True)
        i2 = jnp.min(jnp.where(sim2 == v2, iota, big), axis=1, keepdims=True)
        idx = jnp.concatenate([i1, i2], axis=1)
        oh1 = (iota == i1).astype(jnp.float32)
        oh2 = (iota == i2).astype(jnp.float32)
        bkn1 = jax.lax.dot_general(
            oh1, pn, (((1,), (0,)), ((), ())),
            preferred_element_type=jnp.float32)
        bkn2 = jax.lax.dot_general(
            oh2, pn, (((1,), (0,)), ((), ())),
            preferred_element_type=jnp.float32)
        idx_ref[...] = idx
        sim_ref[...] = sim
        bkn_ref[...] = jnp.concatenate(
            [bkn1[:, None, :], bkn2[:, None, :]], axis=1)
        rs_ref[...] = ((jnp.sum(v1) + jnp.sum(v2)) / batch)[None, None]


def kernel(x_embed, prompt_key):
    batch, seq, d_model = x_embed.shape
    pool = prompt_key.shape[0]
    topk = 2
    hs = seq // 2

    out = pl.pallas_call(
        functools.partial(_body, batch=batch, pool=pool, topk=topk,
                          d_model=d_model),
        grid=(batch,),
        in_specs=[
            pl.BlockSpec((1, hs, d_model), lambda b: (b, 0, 0)),
            pl.BlockSpec((1, hs, d_model), lambda b: (b, 1, 0)),
            pl.BlockSpec((pool, d_model), lambda b: (0, 0)),
        ],
        out_specs=[
            pl.BlockSpec((batch, topk), lambda b: (0, 0)),
            pl.BlockSpec((batch, pool), lambda b: (0, 0)),
            pl.BlockSpec((batch, topk, d_model), lambda b: (0, 0, 0)),
            pl.BlockSpec((1, 1), lambda b: (0, 0)),
        ],
        out_shape=[
            jax.ShapeDtypeStruct((batch, topk), jnp.int32),
            jax.ShapeDtypeStruct((batch, pool), jnp.float32),
            jax.ShapeDtypeStruct((batch, topk, d_model), jnp.float32),
            jax.ShapeDtypeStruct((1, 1), jnp.float32),
        ],
        scratch_shapes=[pltpu.VMEM((max(batch, 8), d_model), jnp.float32)],
    )(x_embed, x_embed, prompt_key)

    idx, sim, bkn, rs = out
    return (idx, sim, bkn, rs.reshape(()))
